# baseline (speedup 1.0000x reference)
import functools

import jax
import jax.numpy as jnp
from jax import lax
from jax.experimental import pallas as pl
from jax.experimental.pallas import tpu as pltpu
from jax.experimental.pallas import tpu_sc as plsc

PATCH_SIZE = 16
SEARCH_SIZE = 384

_NUM_TILES = 32
_LANES = 16
_HALF = 288


def _make_sc_reduce(num_mats, seq, num_patches, tc_mats):
    all_per_tile = num_mats // _NUM_TILES
    own_mats = num_mats - tc_mats
    own_per_tile = own_mats // _NUM_TILES
    assert own_per_tile * _NUM_TILES == own_mats
    full_chunks = (seq // 128) * 8
    mesh = plsc.VectorSubcoreMesh(core_axis_name="c", subcore_axis_name="s")

    @functools.partial(
        pl.kernel,
        mesh=mesh,
        out_type=[
            jax.ShapeDtypeStruct((_NUM_TILES, _LANES), jnp.float32),
            jax.ShapeDtypeStruct((_LANES,), jnp.float32),
        ],
        scratch_types=[
            pltpu.VMEM((_LANES,), jnp.int32),
            pltpu.VMEM((_HALF, 128), jnp.float32),
            pltpu.VMEM((_HALF, 128), jnp.float32),
            pltpu.VMEM((_LANES,), jnp.float32),
            pltpu.SemaphoreType.DMA,
            pltpu.SemaphoreType.DMA,
        ],
    )
    def sc_reduce(attn_hbm, bbox_hbm, out_hbm, cnt_hbm, bbox_v, slab_a,
                  slab_b, acc_v, sem_a, sem_b):
        wid = lax.axis_index("s") * 2 + lax.axis_index("c")
        pltpu.sync_copy(bbox_hbm, bbox_v)
        bb = bbox_v[...]
        x, y, w, h = bb[0], bb[1], bb[2], bb[3]
        i_lo = jnp.maximum(0, y >> 4)
        i_hi = jnp.minimum(num_patches, (y + h + PATCH_SIZE - 1) >> 4)
        j_lo = jnp.maximum(0, x >> 4)
        j_hi = jnp.minimum(num_patches, (x + w + PATCH_SIZE - 1) >> 4)
        lane = jax.lax.iota(jnp.int32, _LANES)
        jt0 = jnp.clip((i_lo * num_patches + j_lo) >> 7, 0, (seq // 128) - 1)

        slabs = (slab_a, slab_b)
        sems = (sem_a, sem_b)

        def piece_src(c_jtile, mat, half):
            r0 = half * _HALF
            col0 = pl.multiple_of(c_jtile * 128, 128)
            return attn_hbm.at[mat, pl.ds(r0, _HALF), pl.ds(col0, 128)]

        def reduce_half(buf, off):
            def red_body(i, accs):
                a0, a1, a2, a3 = accs
                b = i * 4
                return (
                    a0 + buf[b, pl.ds(off, _LANES)],
                    a1 + buf[b + 1, pl.ds(off, _LANES)],
                    a2 + buf[b + 2, pl.ds(off, _LANES)],
                    a3 + buf[b + 3, pl.ds(off, _LANES)],
                )

            z = jnp.zeros((_LANES,), jnp.float32)
            a0, a1, a2, a3 = lax.fori_loop(0, _HALF // 4, red_body,
                                           (z, z, z, z), unroll=4)
            return (a0 + a1) + (a2 + a3)

        def gather_mats(jtile, off, m, mat_list):
            pieces = [(mat, half) for mat in mat_list for half in (0, 1)]
            handles = [pltpu.async_copy(
                piece_src(jtile, pieces[0][0], pieces[0][1]), slabs[0],
                sems[0])]
            csum = jnp.zeros((_LANES,), jnp.float32)
            for j in range(len(pieces)):
                if j + 1 < len(pieces):
                    handles.append(
                        pltpu.async_copy(
                            piece_src(jtile, pieces[j + 1][0],
                                      pieces[j + 1][1]),
                            slabs[(j + 1) % 2], sems[(j + 1) % 2])
                    )
                handles[j].wait()
                csum = csum + reduce_half(slabs[j % 2], off)
            acc_v[...] = acc_v[...] + csum * m

        def seg_hit(ai, rlo, rhi):
            row_ok = jnp.logical_and(ai >= i_lo, ai < i_hi)
            seg_ok = jnp.maximum(rlo, j_lo) < jnp.minimum(rhi, j_hi)
            return jnp.logical_and(row_ok, seg_ok)

        acc_v[...] = jnp.zeros((_LANES,), jnp.float32)

        def chunk_body(c, carry):
            pi, pj, a_i, a_r = carry
            m_bool = (pi >= i_lo) & (pi < i_hi) & (pj >= j_lo) & (pj < j_hi)
            m = jnp.where(
                m_bool,
                jnp.zeros((_LANES,), jnp.float32) + 1.0,
                jnp.zeros((_LANES,), jnp.float32),
            )
            hit1 = seg_hit(a_i, a_r, jnp.minimum(num_patches, a_r + _LANES))
            hit2 = jnp.logical_and(
                a_r + _LANES > num_patches,
                seg_hit(a_i + 1, 0, a_r + _LANES - num_patches),
            )
            active = jnp.logical_or(hit1, hit2)
            jtile = c // 8
            off = (c - jtile * 8) * _LANES
            in_tc_tile = jtile == jt0

            @pl.when(jnp.logical_and(active, in_tc_tile))
            def _():
                gather_mats(
                    jtile, off, m,
                    [tc_mats + wid * own_per_tile + k
                     for k in range(own_per_tile)],
                )

            @pl.when(jnp.logical_and(active, jnp.logical_not(in_tc_tile)))
            def _():
                gather_mats(
                    jtile, off, m,
                    [wid * all_per_tile + k for k in range(all_per_tile)],
                )

            pj2 = pj + _LANES
            wrap = pj2 >= num_patches
            pj2 = jnp.where(wrap, pj2 - num_patches, pj2)
            pi2 = jnp.where(wrap, pi + 1, pi)
            a_r2 = a_r + _LANES
            awrap = a_r2 >= num_patches
            a_r2 = jnp.where(awrap, a_r2 - num_patches, a_r2)
            a_i2 = jnp.where(awrap, a_i + 1, a_i)
            return (pi2, pj2, a_i2, a_r2)

        lax.fori_loop(
            0, full_chunks, chunk_body,
            (jnp.zeros((_LANES,), jnp.int32), lane,
             jnp.zeros((), jnp.int32), jnp.zeros((), jnp.int32)),
        )

        pltpu.sync_copy(acc_v, out_hbm.at[wid])

        @pl.when(wid == 0)
        def _():
            count = jnp.maximum(0, i_hi - i_lo) * jnp.maximum(0, j_hi - j_lo)
            acc_v[...] = (jnp.zeros((_LANES,), jnp.int32) + count).astype(
                jnp.float32
            )
            pltpu.sync_copy(acc_v, cnt_hbm)

    return sc_reduce


def _make_tc_reduce(tc_mats, seq, num_patches):
    mats_per_step = 8
    steps = tc_mats // mats_per_step
    n_jt = seq // 128

    def _bounds(bb):
        x, y, w, h = bb[0], bb[1], bb[2], bb[3]
        i_lo = jnp.maximum(0, y // PATCH_SIZE)
        i_hi = jnp.minimum(num_patches, (y + h + PATCH_SIZE - 1) // PATCH_SIZE)
        j_lo = jnp.maximum(0, x // PATCH_SIZE)
        j_hi = jnp.minimum(num_patches, (x + w + PATCH_SIZE - 1) // PATCH_SIZE)
        return i_lo, i_hi, j_lo, j_hi

    def _jt0(bb):
        i_lo, _, j_lo, _ = _bounds(bb)
        return jnp.clip((i_lo * num_patches + j_lo) // 128, 0, n_jt - 1)

    def tc_body(bb_ref, attn_ref, o_ref):
        i_lo, i_hi, j_lo, j_hi = _bounds(bb_ref)
        jt0 = _jt0(bb_ref)
        col = jt0 * 128 + jax.lax.broadcasted_iota(jnp.int32, (1, 128), 1)
        pi = (col * 2731) >> 16
        pj = col - pi * num_patches
        mask = jnp.where(
            (pi >= i_lo) & (pi < i_hi) & (pj >= j_lo) & (pj < j_hi),
            jnp.float32(1.0), jnp.float32(0.0),
        )
        colsums = jnp.sum(attn_ref[...], axis=(0, 1))
        o_ref[pl.program_id(0), 0] = jnp.sum(colsums * mask[0])

    return pl.pallas_call(
        tc_body,
        grid_spec=pltpu.PrefetchScalarGridSpec(
            num_scalar_prefetch=1,
            grid=(steps,),
            in_specs=[
                pl.BlockSpec((mats_per_step, seq, 128),
                             lambda m, bb: (m, 0, _jt0(bb))),
            ],
            out_specs=pl.BlockSpec(
                (steps, 1), lambda m, bb: (0, 0), memory_space=pltpu.SMEM
            ),
        ),
        out_shape=jax.ShapeDtypeStruct((steps, 1), jnp.float32),
    )


def kernel(attn_weights, bbox):
    num_patches = SEARCH_SIZE // PATCH_SIZE
    search_seq_len = num_patches**2

    num_layers, batch, heads, seq, seq2 = attn_weights.shape
    assert seq == search_seq_len and seq2 == search_seq_len
    num_mats = num_layers * batch * heads
    rows_total = num_mats * seq
    tc_mats = (num_mats // 3) * 2

    attn3 = attn_weights.reshape(num_mats, seq, seq)
    bbox_pad = jnp.zeros((_LANES,), jnp.int32).at[:4].set(bbox)

    partials, cntv = _make_sc_reduce(num_mats, seq, num_patches, tc_mats)(
        attn3, bbox_pad
    )
    tc_sums = _make_tc_reduce(tc_mats, seq, num_patches)(bbox, attn3)

    total = jnp.sum(
        jnp.concatenate([partials.reshape(-1), tc_sums.reshape(-1)])
    )
    count = cntv[0]
    safe_count = jnp.maximum(count, jnp.float32(1.0))

    x, y, w, h = bbox[0], bbox[1], bbox[2], bbox[3]
    i_lo = jnp.maximum(0, y // PATCH_SIZE)
    i_hi = jnp.minimum(num_patches, (y + h + PATCH_SIZE - 1) // PATCH_SIZE)
    j_lo = jnp.maximum(0, x // PATCH_SIZE)
    j_hi = jnp.minimum(num_patches, (x + w + PATCH_SIZE - 1) // PATCH_SIZE)

    tail0 = (seq // 128) * 128
    max_col = (i_hi - 1) * num_patches + (j_hi - 1)
    has_tail = jnp.logical_and(count > 0, max_col >= tail0)

    def _tail_sum(_):
        p = jnp.arange(tail0, search_seq_len, dtype=jnp.int32)
        pi = p // num_patches
        pj = p % num_patches
        tail_mask = (
            (pi >= i_lo) & (pi < i_hi) & (pj >= j_lo) & (pj < j_hi)
        ).astype(jnp.float32)
        return jnp.einsum(
            "mrk,k->", attn3[:, :, tail0:], tail_mask,
            preferred_element_type=jnp.float32,
        )

    tail_total = lax.cond(has_tail, _tail_sum, lambda _: jnp.float32(0.0), 0)

    loss = (total + tail_total) / (safe_count * jnp.float32(rows_total))
    return jnp.where(count == 0, jnp.float32(0.0), loss)


# speedup vs baseline: 1.0763x; 1.0763x over previous
import functools

import jax
import jax.numpy as jnp
from jax import lax
from jax.experimental import pallas as pl
from jax.experimental.pallas import tpu as pltpu
from jax.experimental.pallas import tpu_sc as plsc

PATCH_SIZE = 16
SEARCH_SIZE = 384

_NUM_TILES = 32
_LANES = 16
_HALF = 288


def _make_sc_reduce(num_mats, seq, num_patches, tc_mats):
    all_per_tile = num_mats // _NUM_TILES
    own_mats = num_mats - tc_mats
    own_halves = own_mats * 2 // _NUM_TILES
    assert own_halves * _NUM_TILES == own_mats * 2
    full_chunks = (seq // 128) * 8
    mesh = plsc.VectorSubcoreMesh(core_axis_name="c", subcore_axis_name="s")

    @functools.partial(
        pl.kernel,
        mesh=mesh,
        out_type=[
            jax.ShapeDtypeStruct((_NUM_TILES, _LANES), jnp.float32),
            jax.ShapeDtypeStruct((_LANES,), jnp.float32),
        ],
        scratch_types=[
            pltpu.VMEM((_LANES,), jnp.int32),
            pltpu.VMEM((_HALF, 128), jnp.float32),
            pltpu.VMEM((_HALF, 128), jnp.float32),
            pltpu.VMEM((_LANES,), jnp.float32),
            pltpu.SemaphoreType.DMA,
            pltpu.SemaphoreType.DMA,
        ],
    )
    def sc_reduce(attn_hbm, bbox_hbm, out_hbm, cnt_hbm, bbox_v, slab_a,
                  slab_b, acc_v, sem_a, sem_b):
        wid = lax.axis_index("s") * 2 + lax.axis_index("c")
        pltpu.sync_copy(bbox_hbm, bbox_v)
        bb = bbox_v[...]
        x, y, w, h = bb[0], bb[1], bb[2], bb[3]
        i_lo = jnp.maximum(0, y >> 4)
        i_hi = jnp.minimum(num_patches, (y + h + PATCH_SIZE - 1) >> 4)
        j_lo = jnp.maximum(0, x >> 4)
        j_hi = jnp.minimum(num_patches, (x + w + PATCH_SIZE - 1) >> 4)
        lane = jax.lax.iota(jnp.int32, _LANES)
        jt0 = jnp.clip((i_lo * num_patches + j_lo) >> 7, 0, (seq // 128) - 1)

        slabs = (slab_a, slab_b)
        sems = (sem_a, sem_b)

        def piece_src(c_jtile, mat, half):
            r0 = half * _HALF
            col0 = pl.multiple_of(c_jtile * 128, 128)
            return attn_hbm.at[mat, pl.ds(r0, _HALF), pl.ds(col0, 128)]

        def reduce_half(buf, off):
            def red_body(i, accs):
                a0, a1, a2, a3 = accs
                b = i * 4
                return (
                    a0 + buf[b, pl.ds(off, _LANES)],
                    a1 + buf[b + 1, pl.ds(off, _LANES)],
                    a2 + buf[b + 2, pl.ds(off, _LANES)],
                    a3 + buf[b + 3, pl.ds(off, _LANES)],
                )

            z = jnp.zeros((_LANES,), jnp.float32)
            a0, a1, a2, a3 = lax.fori_loop(0, _HALF // 4, red_body,
                                           (z, z, z, z), unroll=4)
            return (a0 + a1) + (a2 + a3)

        def gather_pieces(jtile, off, m, pieces):
            handles = [pltpu.async_copy(
                piece_src(jtile, pieces[0][0], pieces[0][1]), slabs[0],
                sems[0])]
            csum = jnp.zeros((_LANES,), jnp.float32)
            for j in range(len(pieces)):
                if j + 1 < len(pieces):
                    handles.append(
                        pltpu.async_copy(
                            piece_src(jtile, pieces[j + 1][0],
                                      pieces[j + 1][1]),
                            slabs[(j + 1) % 2], sems[(j + 1) % 2])
                    )
                handles[j].wait()
                csum = csum + reduce_half(slabs[j % 2], off)
            acc_v[...] = acc_v[...] + csum * m

        def seg_hit(ai, rlo, rhi):
            row_ok = jnp.logical_and(ai >= i_lo, ai < i_hi)
            seg_ok = jnp.maximum(rlo, j_lo) < jnp.minimum(rhi, j_hi)
            return jnp.logical_and(row_ok, seg_ok)

        acc_v[...] = jnp.zeros((_LANES,), jnp.float32)

        def chunk_body(c, carry):
            pi, pj, a_i, a_r = carry
            m_bool = (pi >= i_lo) & (pi < i_hi) & (pj >= j_lo) & (pj < j_hi)
            m = jnp.where(
                m_bool,
                jnp.zeros((_LANES,), jnp.float32) + 1.0,
                jnp.zeros((_LANES,), jnp.float32),
            )
            hit1 = seg_hit(a_i, a_r, jnp.minimum(num_patches, a_r + _LANES))
            hit2 = jnp.logical_and(
                a_r + _LANES > num_patches,
                seg_hit(a_i + 1, 0, a_r + _LANES - num_patches),
            )
            active = jnp.logical_or(hit1, hit2)
            jtile = c // 8
            off = (c - jtile * 8) * _LANES
            in_tc_tile = jtile == jt0

            @pl.when(jnp.logical_and(active, in_tc_tile))
            def _():
                gather_pieces(
                    jtile, off, m,
                    [(tc_mats + (wid * own_halves + k) // 2,
                      (wid * own_halves + k) % 2)
                     for k in range(own_halves)],
                )

            @pl.when(jnp.logical_and(active, jnp.logical_not(in_tc_tile)))
            def _():
                gather_pieces(
                    jtile, off, m,
                    [(wid * all_per_tile + k, half)
                     for k in range(all_per_tile) for half in (0, 1)],
                )

            pj2 = pj + _LANES
            wrap = pj2 >= num_patches
            pj2 = jnp.where(wrap, pj2 - num_patches, pj2)
            pi2 = jnp.where(wrap, pi + 1, pi)
            a_r2 = a_r + _LANES
            awrap = a_r2 >= num_patches
            a_r2 = jnp.where(awrap, a_r2 - num_patches, a_r2)
            a_i2 = jnp.where(awrap, a_i + 1, a_i)
            return (pi2, pj2, a_i2, a_r2)

        lax.fori_loop(
            0, full_chunks, chunk_body,
            (jnp.zeros((_LANES,), jnp.int32), lane,
             jnp.zeros((), jnp.int32), jnp.zeros((), jnp.int32)),
        )

        pltpu.sync_copy(acc_v, out_hbm.at[wid])

        @pl.when(wid == 0)
        def _():
            count = jnp.maximum(0, i_hi - i_lo) * jnp.maximum(0, j_hi - j_lo)
            acc_v[...] = (jnp.zeros((_LANES,), jnp.int32) + count).astype(
                jnp.float32
            )
            pltpu.sync_copy(acc_v, cnt_hbm)

    return sc_reduce


def _make_tc_reduce(tc_mats, seq, num_patches):
    mats_per_step = 8
    steps = tc_mats // mats_per_step
    n_jt = seq // 128

    def _bounds(bb):
        x, y, w, h = bb[0], bb[1], bb[2], bb[3]
        i_lo = jnp.maximum(0, y // PATCH_SIZE)
        i_hi = jnp.minimum(num_patches, (y + h + PATCH_SIZE - 1) // PATCH_SIZE)
        j_lo = jnp.maximum(0, x // PATCH_SIZE)
        j_hi = jnp.minimum(num_patches, (x + w + PATCH_SIZE - 1) // PATCH_SIZE)
        return i_lo, i_hi, j_lo, j_hi

    def _jt0(bb):
        i_lo, _, j_lo, _ = _bounds(bb)
        return jnp.clip((i_lo * num_patches + j_lo) // 128, 0, n_jt - 1)

    def tc_body(bb_ref, attn_ref, o_ref):
        i_lo, i_hi, j_lo, j_hi = _bounds(bb_ref)
        jt0 = _jt0(bb_ref)
        col = jt0 * 128 + jax.lax.broadcasted_iota(jnp.int32, (1, 128), 1)
        pi = (col * 2731) >> 16
        pj = col - pi * num_patches
        mask = jnp.where(
            (pi >= i_lo) & (pi < i_hi) & (pj >= j_lo) & (pj < j_hi),
            jnp.float32(1.0), jnp.float32(0.0),
        )
        colsums = jnp.sum(attn_ref[...], axis=(0, 1))
        o_ref[pl.program_id(0), 0] = jnp.sum(colsums * mask[0])

    return pl.pallas_call(
        tc_body,
        grid_spec=pltpu.PrefetchScalarGridSpec(
            num_scalar_prefetch=1,
            grid=(steps,),
            in_specs=[
                pl.BlockSpec((mats_per_step, seq, 128),
                             lambda m, bb: (m, 0, _jt0(bb))),
            ],
            out_specs=pl.BlockSpec(
                (steps, 1), lambda m, bb: (0, 0), memory_space=pltpu.SMEM
            ),
        ),
        out_shape=jax.ShapeDtypeStruct((steps, 1), jnp.float32),
    )


def kernel(attn_weights, bbox):
    num_patches = SEARCH_SIZE // PATCH_SIZE
    search_seq_len = num_patches**2

    num_layers, batch, heads, seq, seq2 = attn_weights.shape
    assert seq == search_seq_len and seq2 == search_seq_len
    num_mats = num_layers * batch * heads
    rows_total = num_mats * seq
    tc_mats = num_mats // 2

    attn3 = attn_weights.reshape(num_mats, seq, seq)
    bbox_pad = jnp.zeros((_LANES,), jnp.int32).at[:4].set(bbox)

    partials, cntv = _make_sc_reduce(num_mats, seq, num_patches, tc_mats)(
        attn3, bbox_pad
    )
    tc_sums = _make_tc_reduce(tc_mats, seq, num_patches)(bbox, attn3)

    total = jnp.sum(
        jnp.concatenate([partials.reshape(-1), tc_sums.reshape(-1)])
    )
    count = cntv[0]
    safe_count = jnp.maximum(count, jnp.float32(1.0))

    x, y, w, h = bbox[0], bbox[1], bbox[2], bbox[3]
    i_lo = jnp.maximum(0, y // PATCH_SIZE)
    i_hi = jnp.minimum(num_patches, (y + h + PATCH_SIZE - 1) // PATCH_SIZE)
    j_lo = jnp.maximum(0, x // PATCH_SIZE)
    j_hi = jnp.minimum(num_patches, (x + w + PATCH_SIZE - 1) // PATCH_SIZE)

    tail0 = (seq // 128) * 128
    max_col = (i_hi - 1) * num_patches + (j_hi - 1)
    has_tail = jnp.logical_and(count > 0, max_col >= tail0)

    def _tail_sum(_):
        p = jnp.arange(tail0, search_seq_len, dtype=jnp.int32)
        pi = p // num_patches
        pj = p % num_patches
        tail_mask = (
            (pi >= i_lo) & (pi < i_hi) & (pj >= j_lo) & (pj < j_hi)
        ).astype(jnp.float32)
        return jnp.einsum(
            "mrk,k->", attn3[:, :, tail0:], tail_mask,
            preferred_element_type=jnp.float32,
        )

    tail_total = lax.cond(has_tail, _tail_sum, lambda _: jnp.float32(0.0), 0)

    loss = (total + tail_total) / (safe_count * jnp.float32(rows_total))
    return jnp.where(count == 0, jnp.float32(0.0), loss)


# speedup vs baseline: 1.0828x; 1.0061x over previous
import functools

import jax
import jax.numpy as jnp
from jax import lax
from jax.experimental import pallas as pl
from jax.experimental.pallas import tpu as pltpu
from jax.experimental.pallas import tpu_sc as plsc

PATCH_SIZE = 16
SEARCH_SIZE = 384

_NUM_TILES = 32
_LANES = 16
_HALF = 288


def _make_sc_reduce(num_mats, seq, num_patches, tc_mats):
    all_per_tile = num_mats // _NUM_TILES
    own_mats = num_mats - tc_mats
    own_halves = own_mats * 2 // _NUM_TILES
    assert own_halves * _NUM_TILES == own_mats * 2
    full_chunks = (seq // 128) * 8
    mesh = plsc.VectorSubcoreMesh(core_axis_name="c", subcore_axis_name="s")

    @functools.partial(
        pl.kernel,
        mesh=mesh,
        out_type=[
            jax.ShapeDtypeStruct((_NUM_TILES, _LANES), jnp.float32),
            jax.ShapeDtypeStruct((_LANES,), jnp.float32),
        ],
        scratch_types=[
            pltpu.VMEM((_LANES,), jnp.int32),
            pltpu.VMEM((_HALF, 128), jnp.float32),
            pltpu.VMEM((_HALF, 128), jnp.float32),
            pltpu.VMEM((_LANES,), jnp.float32),
            pltpu.SemaphoreType.DMA,
            pltpu.SemaphoreType.DMA,
        ],
    )
    def sc_reduce(attn_hbm, bbox_hbm, out_hbm, cnt_hbm, bbox_v, slab_a,
                  slab_b, acc_v, sem_a, sem_b):
        wid = lax.axis_index("s") * 2 + lax.axis_index("c")
        pltpu.sync_copy(bbox_hbm, bbox_v.at[pl.ds(0, 4)])
        bb = bbox_v[...]
        x, y, w, h = bb[0], bb[1], bb[2], bb[3]
        i_lo = jnp.maximum(0, y >> 4)
        i_hi = jnp.minimum(num_patches, (y + h + PATCH_SIZE - 1) >> 4)
        j_lo = jnp.maximum(0, x >> 4)
        j_hi = jnp.minimum(num_patches, (x + w + PATCH_SIZE - 1) >> 4)
        lane = jax.lax.iota(jnp.int32, _LANES)
        jt0 = jnp.clip((i_lo * num_patches + j_lo) >> 7, 0, (seq // 128) - 1)

        slabs = (slab_a, slab_b)
        sems = (sem_a, sem_b)

        def piece_src(c_jtile, mat, half):
            r0 = half * _HALF
            col0 = pl.multiple_of(c_jtile * 128, 128)
            return attn_hbm.at[mat, pl.ds(r0, _HALF), pl.ds(col0, 128)]

        def reduce_half(buf, off):
            def red_body(i, accs):
                a0, a1, a2, a3 = accs
                b = i * 4
                return (
                    a0 + buf[b, pl.ds(off, _LANES)],
                    a1 + buf[b + 1, pl.ds(off, _LANES)],
                    a2 + buf[b + 2, pl.ds(off, _LANES)],
                    a3 + buf[b + 3, pl.ds(off, _LANES)],
                )

            z = jnp.zeros((_LANES,), jnp.float32)
            a0, a1, a2, a3 = lax.fori_loop(0, _HALF // 4, red_body,
                                           (z, z, z, z))
            return (a0 + a1) + (a2 + a3)

        def gather_pieces(jtile, off, m, pieces):
            handles = [pltpu.async_copy(
                piece_src(jtile, pieces[0][0], pieces[0][1]), slabs[0],
                sems[0])]
            csum = jnp.zeros((_LANES,), jnp.float32)
            for j in range(len(pieces)):
                if j + 1 < len(pieces):
                    handles.append(
                        pltpu.async_copy(
                            piece_src(jtile, pieces[j + 1][0],
                                      pieces[j + 1][1]),
                            slabs[(j + 1) % 2], sems[(j + 1) % 2])
                    )
                handles[j].wait()
                csum = csum + reduce_half(slabs[j % 2], off)
            acc_v[...] = acc_v[...] + csum * m

        def seg_hit(ai, rlo, rhi):
            row_ok = jnp.logical_and(ai >= i_lo, ai < i_hi)
            seg_ok = jnp.maximum(rlo, j_lo) < jnp.minimum(rhi, j_hi)
            return jnp.logical_and(row_ok, seg_ok)

        acc_v[...] = jnp.zeros((_LANES,), jnp.float32)

        def chunk_body(c, carry):
            pi, pj, a_i, a_r = carry
            m_bool = (pi >= i_lo) & (pi < i_hi) & (pj >= j_lo) & (pj < j_hi)
            m = jnp.where(
                m_bool,
                jnp.zeros((_LANES,), jnp.float32) + 1.0,
                jnp.zeros((_LANES,), jnp.float32),
            )
            hit1 = seg_hit(a_i, a_r, jnp.minimum(num_patches, a_r + _LANES))
            hit2 = jnp.logical_and(
                a_r + _LANES > num_patches,
                seg_hit(a_i + 1, 0, a_r + _LANES - num_patches),
            )
            active = jnp.logical_or(hit1, hit2)
            jtile = c // 8
            off = (c - jtile * 8) * _LANES
            in_tc_tile = jtile == jt0

            @pl.when(jnp.logical_and(active, in_tc_tile))
            def _():
                gather_pieces(
                    jtile, off, m,
                    [(tc_mats + (wid * own_halves + k) // 2,
                      (wid * own_halves + k) % 2)
                     for k in range(own_halves)],
                )

            @pl.when(jnp.logical_and(active, jnp.logical_not(in_tc_tile)))
            def _():
                gather_pieces(
                    jtile, off, m,
                    [(wid * all_per_tile + k, half)
                     for k in range(all_per_tile) for half in (0, 1)],
                )

            pj2 = pj + _LANES
            wrap = pj2 >= num_patches
            pj2 = jnp.where(wrap, pj2 - num_patches, pj2)
            pi2 = jnp.where(wrap, pi + 1, pi)
            a_r2 = a_r + _LANES
            awrap = a_r2 >= num_patches
            a_r2 = jnp.where(awrap, a_r2 - num_patches, a_r2)
            a_i2 = jnp.where(awrap, a_i + 1, a_i)
            return (pi2, pj2, a_i2, a_r2)

        lax.fori_loop(
            0, full_chunks, chunk_body,
            (jnp.zeros((_LANES,), jnp.int32), lane,
             jnp.zeros((), jnp.int32), jnp.zeros((), jnp.int32)),
        )

        pltpu.sync_copy(acc_v, out_hbm.at[wid])

        @pl.when(wid == 0)
        def _():
            count = jnp.maximum(0, i_hi - i_lo) * jnp.maximum(0, j_hi - j_lo)
            acc_v[...] = (jnp.zeros((_LANES,), jnp.int32) + count).astype(
                jnp.float32
            )
            pltpu.sync_copy(acc_v, cnt_hbm)

    return sc_reduce


def _make_tc_reduce(tc_mats, seq, num_patches):
    mats_per_step = 8
    steps = tc_mats // mats_per_step
    n_jt = seq // 128

    def _bounds(bb):
        x, y, w, h = bb[0], bb[1], bb[2], bb[3]
        i_lo = jnp.maximum(0, y // PATCH_SIZE)
        i_hi = jnp.minimum(num_patches, (y + h + PATCH_SIZE - 1) // PATCH_SIZE)
        j_lo = jnp.maximum(0, x // PATCH_SIZE)
        j_hi = jnp.minimum(num_patches, (x + w + PATCH_SIZE - 1) // PATCH_SIZE)
        return i_lo, i_hi, j_lo, j_hi

    def _jt0(bb):
        i_lo, _, j_lo, _ = _bounds(bb)
        return jnp.clip((i_lo * num_patches + j_lo) // 128, 0, n_jt - 1)

    def tc_body(bb_ref, attn_ref, o_ref):
        i_lo, i_hi, j_lo, j_hi = _bounds(bb_ref)
        jt0 = _jt0(bb_ref)
        col = jt0 * 128 + jax.lax.broadcasted_iota(jnp.int32, (1, 128), 1)
        pi = (col * 2731) >> 16
        pj = col - pi * num_patches
        mask = jnp.where(
            (pi >= i_lo) & (pi < i_hi) & (pj >= j_lo) & (pj < j_hi),
            jnp.float32(1.0), jnp.float32(0.0),
        )
        colsums = jnp.sum(attn_ref[...], axis=(0, 1))
        o_ref[pl.program_id(0), 0] = jnp.sum(colsums * mask[0])

    return pl.pallas_call(
        tc_body,
        grid_spec=pltpu.PrefetchScalarGridSpec(
            num_scalar_prefetch=1,
            grid=(steps,),
            in_specs=[
                pl.BlockSpec((mats_per_step, seq, 128),
                             lambda m, bb: (m, 0, _jt0(bb))),
            ],
            out_specs=pl.BlockSpec(
                (steps, 1), lambda m, bb: (0, 0), memory_space=pltpu.SMEM
            ),
        ),
        out_shape=jax.ShapeDtypeStruct((steps, 1), jnp.float32),
    )


def kernel(attn_weights, bbox):
    num_patches = SEARCH_SIZE // PATCH_SIZE
    search_seq_len = num_patches**2

    num_layers, batch, heads, seq, seq2 = attn_weights.shape
    assert seq == search_seq_len and seq2 == search_seq_len
    num_mats = num_layers * batch * heads
    rows_total = num_mats * seq
    tc_mats = num_mats // 2

    attn3 = attn_weights.reshape(num_mats, seq, seq)

    partials, cntv = _make_sc_reduce(num_mats, seq, num_patches, tc_mats)(
        attn3, bbox
    )
    tc_sums = _make_tc_reduce(tc_mats, seq, num_patches)(bbox, attn3)

    total = jnp.sum(
        jnp.concatenate([partials.reshape(-1), tc_sums.reshape(-1)])
    )
    count = cntv[0]
    safe_count = jnp.maximum(count, jnp.float32(1.0))

    x, y, w, h = bbox[0], bbox[1], bbox[2], bbox[3]
    i_lo = jnp.maximum(0, y // PATCH_SIZE)
    i_hi = jnp.minimum(num_patches, (y + h + PATCH_SIZE - 1) // PATCH_SIZE)
    j_lo = jnp.maximum(0, x // PATCH_SIZE)
    j_hi = jnp.minimum(num_patches, (x + w + PATCH_SIZE - 1) // PATCH_SIZE)

    tail0 = (seq // 128) * 128
    max_col = (i_hi - 1) * num_patches + (j_hi - 1)
    has_tail = jnp.logical_and(count > 0, max_col >= tail0)

    def _tail_sum(_):
        p = jnp.arange(tail0, search_seq_len, dtype=jnp.int32)
        pi = p // num_patches
        pj = p % num_patches
        tail_mask = (
            (pi >= i_lo) & (pi < i_hi) & (pj >= j_lo) & (pj < j_hi)
        ).astype(jnp.float32)
        return jnp.einsum(
            "mrk,k->", attn3[:, :, tail0:], tail_mask,
            preferred_element_type=jnp.float32,
        )

    tail_total = lax.cond(has_tail, _tail_sum, lambda _: jnp.float32(0.0), 0)

    loss = (total + tail_total) / (safe_count * jnp.float32(rows_total))
    return jnp.where(count == 0, jnp.float32(0.0), loss)
